# baseline (speedup 1.0000x reference)
import numpy as np
import jax
import jax.numpy as jnp
from jax.experimental import pallas as pl
from jax.experimental.pallas import tpu as pltpu


def _peo_np():
    m = np.zeros((128, 128), np.float32)
    for j in range(8):
        for c in range(16):
            m[16 * j + c, 64 * (j % 2) + 16 * (j // 2) + c] = 1.0
    return m

H, W = 128, 128
C1, C2, C3 = 16, 32, 64
S2, S3 = 66, 34
M1 = H * W
M2 = 64 * S2
M3 = 32 * S3
A2R = S2 * S2 + 20
A3R = S3 * S3 + 20
HP = 16


def _tower_kernel(p1_ref, w1_ref, b1_ref, w2_ref, b2_ref, w3_ref, b3_ref,
                  peo_ref, eo2_ref, eo3_ref, feat_ref,
                  z1_s, a2_s, p2_s, z2_s, a3_s, p3_s, z3_s):
    f32 = jnp.float32

    z1_s[...] = jnp.dot(p1_ref[0], w1_ref[...], preferred_element_type=f32)

    a2_s[...] = jnp.zeros_like(a2_s)
    a3_s[...] = jnp.zeros_like(a3_s)

    peo = peo_ref[...]
    b1q = jnp.concatenate([b1_ref[...]] * 4, axis=1)
    for py in range(64):
        r0 = 32 * py
        vtp = jnp.maximum(z1_s[pl.ds(r0, 16), :], z1_s[pl.ds(r0 + 16, 16), :])
        pk = jnp.dot(vtp, peo, preferred_element_type=f32)
        m = jnp.maximum(jnp.maximum(pk[:, 0:64], pk[:, 64:128]) + b1q,
                        0.0).astype(a2_s.dtype)
        a2_s[pl.ds((py + 1) * S2 + 1, 64), :] = m.reshape(64, C1)

    for t in range(9):
        ky, kx = divmod(t, 3)
        p2_s[:, C1 * t:C1 * (t + 1)] = a2_s[pl.ds(ky * S2 + kx, M2), :]
    z2_s[...] = jnp.dot(p2_s[...], w2_ref[...], preferred_element_type=f32)

    eo2, b2 = eo2_ref[...], b2_ref[...]
    for py in range(32):
        r0 = 2 * py * S2
        vt = jnp.maximum(z2_s[pl.ds(r0, 64), :], z2_s[pl.ds(r0 + S2, 64), :])
        m2 = jnp.dot(eo2, vt, preferred_element_type=f32)
        m = jnp.maximum(m2[0:32], m2[32:64])
        a3_s[pl.ds((py + 1) * S3 + 1, 32), :] = (
            jnp.maximum(m + b2, 0.0).astype(a3_s.dtype))

    for t in range(9):
        ky, kx = divmod(t, 3)
        p3_s[:, C2 * t:C2 * (t + 1)] = a3_s[pl.ds(ky * S3 + kx, M3), :]
    z3_s[...] = jnp.dot(p3_s[...], w3_ref[...], preferred_element_type=f32)

    eo3, b3 = eo3_ref[...], b3_ref[...]
    for py in range(HP):
        r0 = 2 * py * S3
        vt = jnp.maximum(z3_s[pl.ds(r0, 32), :], z3_s[pl.ds(r0 + S3, 32), :])
        m2 = jnp.dot(eo3, vt, preferred_element_type=f32)
        m = jnp.maximum(m2[0:16], m2[16:32])
        feat_ref[0, pl.ds(py * HP, HP), :] = (
            jnp.maximum(m + b3, 0.0).astype(feat_ref.dtype))


def _conv_tower(p1, w1, b1, w2, b2, w3, b3, eo1, eo2, eo3):
    B = p1.shape[0]
    return pl.pallas_call(
        _tower_kernel,
        out_shape=jax.ShapeDtypeStruct((B, HP * HP, C3), jnp.bfloat16),
        grid=(B,),
        in_specs=[
            pl.BlockSpec((1, M1 // 8, 8 * 27), lambda b: (b, 0, 0)),
            pl.BlockSpec((8 * 27, 8 * C1), lambda b: (0, 0)),
            pl.BlockSpec((1, C1), lambda b: (0, 0)),
            pl.BlockSpec((9 * C1, C2), lambda b: (0, 0)),
            pl.BlockSpec((1, C2), lambda b: (0, 0)),
            pl.BlockSpec((9 * C2, C3), lambda b: (0, 0)),
            pl.BlockSpec((1, C3), lambda b: (0, 0)),
            pl.BlockSpec((128, 128), lambda b: (0, 0)),
            pl.BlockSpec((64, 64), lambda b: (0, 0)),
            pl.BlockSpec((32, 32), lambda b: (0, 0)),
        ],
        out_specs=pl.BlockSpec((1, HP * HP, C3), lambda b: (b, 0, 0)),
        scratch_shapes=[
            pltpu.VMEM((M1 // 8, 8 * C1), jnp.float32),
            pltpu.VMEM((A2R, C1), jnp.bfloat16),
            pltpu.VMEM((M2, 9 * C1), jnp.bfloat16),
            pltpu.VMEM((M2, C2), jnp.float32),
            pltpu.VMEM((A3R, C2), jnp.bfloat16),
            pltpu.VMEM((M3, 9 * C2), jnp.bfloat16),
            pltpu.VMEM((M3, C3), jnp.float32),
        ],
        compiler_params=pltpu.CompilerParams(dimension_semantics=("parallel",)),
    )(p1, w1, b1, w2, b2, w3, b3, eo1, eo2, eo3)


def _mlp_kernel(x_ref, w1_ref, b1_ref, w2_ref, b2_ref, o_ref):
    h = jnp.dot(x_ref[...], w1_ref[...], preferred_element_type=jnp.float32)
    h = jnp.maximum(h + b1_ref[...], 0.0)
    o_ref[...] = (jnp.dot(h, w2_ref[...], preferred_element_type=jnp.float32)
                  + b2_ref[...])


def _classifier(flat, w1, b1, w2, b2):
    B, D = flat.shape
    Hd, C = w1.shape[1], w2.shape[1]
    mb = B // 2
    return pl.pallas_call(
        _mlp_kernel,
        out_shape=jax.ShapeDtypeStruct((B, C), jnp.float32),
        grid=(2,),
        in_specs=[pl.BlockSpec((mb, D), lambda i: (i, 0)),
                  pl.BlockSpec((D, Hd), lambda i: (0, 0)),
                  pl.BlockSpec((1, Hd), lambda i: (0, 0)),
                  pl.BlockSpec((Hd, C), lambda i: (0, 0)),
                  pl.BlockSpec((1, C), lambda i: (0, 0))],
        out_specs=pl.BlockSpec((mb, C), lambda i: (i, 0)),
        compiler_params=pltpu.CompilerParams(dimension_semantics=("parallel",)),
    )(flat, w1, b1, w2, b2)


def kernel(x_nchw, w1t, b1r, w2t, b2r, w3t, b3r,
           fc1_wt, fc1_br, fc2_wt, fc2_br, sel1, sel2, sel3):
    B = x_nchw.shape[0]
    bf16 = jnp.bfloat16

    x = jnp.transpose(x_nchw, (0, 2, 3, 1)).astype(bf16)
    xp = jnp.pad(x, ((0, 0), (1, 1), (1, 1), (0, 0)))
    cols = [xp[:, ky:ky + H, kx:kx + W, :] for ky in range(3) for kx in range(3)]
    p1 = jnp.concatenate(cols, axis=-1).reshape(B, M1 // 8, 8 * 27)

    w1 = jnp.kron(jnp.eye(8, dtype=jnp.float32), w1t).astype(bf16)
    w2 = w2t.reshape(9 * C1, C2).astype(bf16)
    w3 = w3t.reshape(9 * C2, C3).astype(bf16)

    peo = jnp.asarray(_peo_np())
    eo2 = sel2.reshape(64, 64)
    eo3 = sel3.reshape(32, 32)

    feats = _conv_tower(p1, w1, b1r, w2, b2r, w3, b3r, peo, eo2, eo3)
    flat = feats.reshape(B, HP * HP * C3)
    return _classifier(flat, fc1_wt.astype(bf16), fc1_br, fc2_wt, fc2_br)


# speedup vs baseline: 4.1886x; 4.1886x over previous
import numpy as np
import jax
import jax.numpy as jnp
from jax.experimental import pallas as pl
from jax.experimental.pallas import tpu as pltpu


def _peo_np():
    m = np.zeros((128, 128), np.float32)
    for j in range(8):
        for c in range(16):
            m[16 * j + c, 64 * (j % 2) + 16 * (j // 2) + c] = 1.0
    return m

H, W = 128, 128
C1, C2, C3 = 16, 32, 64
S2, S3 = 66, 34
M1 = H * W
M2 = 64 * S2
M3 = 32 * S3
A2R = S2 * S2 + 20
A3R = S3 * S3 + 20
HP = 16


def _tower_kernel(p1_ref, w1_ref, b1_ref, w2_ref, b2_ref, w3_ref, b3_ref,
                  peo_ref, eo2_ref, eo3_ref, feat_ref,
                  z1_s, a2_s, p2_s, z2_s, a3_s, p3_s, z3_s):
    f32 = jnp.float32

    z1_s[...] = jnp.dot(p1_ref[0], w1_ref[...], preferred_element_type=f32)

    a2_s[...] = jnp.zeros_like(a2_s)
    a3_s[...] = jnp.zeros_like(a3_s)

    peo = peo_ref[...]
    b1q = jnp.concatenate([b1_ref[...]] * 4, axis=1)
    for py in range(64):
        r0 = 32 * py
        vtp = jnp.maximum(z1_s[pl.ds(r0, 16), :], z1_s[pl.ds(r0 + 16, 16), :])
        pk = jnp.dot(vtp, peo, preferred_element_type=f32)
        m = jnp.maximum(jnp.maximum(pk[:, 0:64], pk[:, 64:128]) + b1q, 0.0)
        a2_s[pl.ds((py + 1) * S2 + 1, 64), :] = (
            m.reshape(64, C1).astype(a2_s.dtype))

    for t in range(9):
        ky, kx = divmod(t, 3)
        p2_s[:, C1 * t:C1 * (t + 1)] = a2_s[pl.ds(ky * S2 + kx, M2), :]
    z2_s[...] = jnp.dot(p2_s[...], w2_ref[...], preferred_element_type=f32)

    eo2, b2 = eo2_ref[...], b2_ref[...]
    for py in range(32):
        r0 = 2 * py * S2
        vt = jnp.maximum(z2_s[pl.ds(r0, 64), :], z2_s[pl.ds(r0 + S2, 64), :])
        m2 = jnp.dot(eo2, vt, preferred_element_type=f32)
        m = jnp.maximum(m2[0:32], m2[32:64])
        a3_s[pl.ds((py + 1) * S3 + 1, 32), :] = (
            jnp.maximum(m + b2, 0.0).astype(a3_s.dtype))

    for t in range(9):
        ky, kx = divmod(t, 3)
        p3_s[:, C2 * t:C2 * (t + 1)] = a3_s[pl.ds(ky * S3 + kx, M3), :]
    z3_s[...] = jnp.dot(p3_s[...], w3_ref[...], preferred_element_type=f32)

    eo3, b3 = eo3_ref[...], b3_ref[...]
    for py in range(HP):
        r0 = 2 * py * S3
        vt = jnp.maximum(z3_s[pl.ds(r0, 32), :], z3_s[pl.ds(r0 + S3, 32), :])
        m2 = jnp.dot(eo3, vt, preferred_element_type=f32)
        m = jnp.maximum(m2[0:16], m2[16:32])
        feat_ref[0, pl.ds(py * HP, HP), :] = (
            jnp.maximum(m + b3, 0.0).astype(feat_ref.dtype))


def _conv_tower(p1, w1, b1, w2, b2, w3, b3, eo1, eo2, eo3):
    B = p1.shape[0]
    return pl.pallas_call(
        _tower_kernel,
        out_shape=jax.ShapeDtypeStruct((B, HP * HP, C3), jnp.bfloat16),
        grid=(B,),
        in_specs=[
            pl.BlockSpec((1, M1 // 8, 256), lambda b: (b, 0, 0)),
            pl.BlockSpec((256, 8 * C1), lambda b: (0, 0)),
            pl.BlockSpec((1, C1), lambda b: (0, 0)),
            pl.BlockSpec((9 * C1, C2), lambda b: (0, 0)),
            pl.BlockSpec((1, C2), lambda b: (0, 0)),
            pl.BlockSpec((9 * C2, C3), lambda b: (0, 0)),
            pl.BlockSpec((1, C3), lambda b: (0, 0)),
            pl.BlockSpec((128, 128), lambda b: (0, 0)),
            pl.BlockSpec((64, 64), lambda b: (0, 0)),
            pl.BlockSpec((32, 32), lambda b: (0, 0)),
        ],
        out_specs=pl.BlockSpec((1, HP * HP, C3), lambda b: (b, 0, 0)),
        scratch_shapes=[
            pltpu.VMEM((M1 // 8, 8 * C1), jnp.float32),
            pltpu.VMEM((A2R, C1), jnp.bfloat16),
            pltpu.VMEM((M2, 9 * C1), jnp.bfloat16),
            pltpu.VMEM((M2, C2), jnp.float32),
            pltpu.VMEM((A3R, C2), jnp.bfloat16),
            pltpu.VMEM((M3, 9 * C2), jnp.bfloat16),
            pltpu.VMEM((M3, C3), jnp.float32),
        ],
        compiler_params=pltpu.CompilerParams(dimension_semantics=("parallel",)),
    )(p1, w1, b1, w2, b2, w3, b3, eo1, eo2, eo3)


def _mlp_kernel(x_ref, w1_ref, b1_ref, w2_ref, b2_ref, o_ref):
    h = jnp.dot(x_ref[...], w1_ref[...], preferred_element_type=jnp.float32)
    h = jnp.maximum(h + b1_ref[...], 0.0)
    o_ref[...] = (jnp.dot(h, w2_ref[...], preferred_element_type=jnp.float32)
                  + b2_ref[...])


def _classifier(flat, w1, b1, w2, b2):
    B, D = flat.shape
    Hd, C = w1.shape[1], w2.shape[1]
    mb = B // 2
    return pl.pallas_call(
        _mlp_kernel,
        out_shape=jax.ShapeDtypeStruct((B, C), jnp.float32),
        grid=(2,),
        in_specs=[pl.BlockSpec((mb, D), lambda i: (i, 0)),
                  pl.BlockSpec((D, Hd), lambda i: (0, 0)),
                  pl.BlockSpec((1, Hd), lambda i: (0, 0)),
                  pl.BlockSpec((Hd, C), lambda i: (0, 0)),
                  pl.BlockSpec((1, C), lambda i: (0, 0))],
        out_specs=pl.BlockSpec((mb, C), lambda i: (i, 0)),
        compiler_params=pltpu.CompilerParams(dimension_semantics=("parallel",)),
    )(flat, w1, b1, w2, b2)


def kernel(x_nchw, w1t, b1r, w2t, b2r, w3t, b3r,
           fc1_wt, fc1_br, fc2_wt, fc2_br, sel1, sel2, sel3):
    B = x_nchw.shape[0]
    bf16 = jnp.bfloat16

    x = jnp.transpose(x_nchw, (0, 2, 3, 1)).astype(bf16)
    xp = jnp.pad(x, ((0, 0), (1, 1), (1, 1), (0, 0)))
    cols = [xp[:, ky:ky + H, kx:kx + W, :] for ky in range(3) for kx in range(3)]
    cols.append(jnp.zeros((B, H, W, 5), bf16))
    p1 = jnp.concatenate(cols, axis=-1).reshape(B, M1 // 8, 256)

    w1pad = jnp.pad(w1t, ((0, 5), (0, 0)))
    w1 = jnp.kron(jnp.eye(8, dtype=jnp.float32), w1pad).astype(bf16)
    w2 = w2t.reshape(9 * C1, C2).astype(bf16)
    w3 = w3t.reshape(9 * C2, C3).astype(bf16)

    peo = jnp.asarray(_peo_np())
    eo2 = sel2.reshape(64, 64)
    eo3 = sel3.reshape(32, 32)

    feats = _conv_tower(p1, w1, b1r, w2, b2r, w3, b3r, peo, eo2, eo3)
    flat = feats.reshape(B, HP * HP * C3)
    return _classifier(flat, fc1_wt.astype(bf16), fc1_br, fc2_wt, fc2_br)


# speedup vs baseline: 24.7522x; 5.9094x over previous
import jax
import jax.numpy as jnp
from jax.experimental import pallas as pl
from jax.experimental.pallas import tpu as pltpu

H, W = 128, 128
C1, C2, C3 = 16, 32, 64
S2, S3 = 66, 34
M1 = H * W
M2 = 64 * S2
M3 = 32 * S3
A2R = S2 * S2 + 20
A3R = S3 * S3 + 20
HP = 16


def _tower_kernel(p1_ref, w1_ref, b1_ref, w2_ref, b2_ref, w3_ref, b3_ref,
                  eo1_ref, eo2_ref, eo3_ref, feat_ref,
                  z1_s, a2_s, p2_s, z2_s, a3_s, p3_s, z3_s):
    f32 = jnp.float32

    z1_s[...] = jnp.dot(p1_ref[0], w1_ref[...], preferred_element_type=f32)

    a2_s[...] = jnp.zeros_like(a2_s)
    a3_s[...] = jnp.zeros_like(a3_s)

    eo1, b1 = eo1_ref[...], b1_ref[...]
    for py in range(64):
        r0 = 2 * py * W
        vt = jnp.maximum(z1_s[pl.ds(r0, 128), :], z1_s[pl.ds(r0 + W, 128), :])
        m2 = jnp.dot(eo1, vt, preferred_element_type=f32)
        m = jnp.maximum(m2[0:64], m2[64:128])
        a2_s[pl.ds((py + 1) * S2 + 1, 64), :] = (
            jnp.maximum(m + b1, 0.0).astype(a2_s.dtype))

    for t in range(9):
        ky, kx = divmod(t, 3)
        p2_s[:, C1 * t:C1 * (t + 1)] = a2_s[pl.ds(ky * S2 + kx, M2), :]
    z2_s[...] = jnp.dot(p2_s[...], w2_ref[...], preferred_element_type=f32)

    eo2, b2 = eo2_ref[...], b2_ref[...]
    for py in range(32):
        r0 = 2 * py * S2
        vt = jnp.maximum(z2_s[pl.ds(r0, 64), :], z2_s[pl.ds(r0 + S2, 64), :])
        m2 = jnp.dot(eo2, vt, preferred_element_type=f32)
        m = jnp.maximum(m2[0:32], m2[32:64])
        a3_s[pl.ds((py + 1) * S3 + 1, 32), :] = (
            jnp.maximum(m + b2, 0.0).astype(a3_s.dtype))

    for t in range(9):
        ky, kx = divmod(t, 3)
        p3_s[:, C2 * t:C2 * (t + 1)] = a3_s[pl.ds(ky * S3 + kx, M3), :]
    z3_s[...] = jnp.dot(p3_s[...], w3_ref[...], preferred_element_type=f32)

    eo3, b3 = eo3_ref[...], b3_ref[...]
    for py in range(HP):
        r0 = 2 * py * S3
        vt = jnp.maximum(z3_s[pl.ds(r0, 32), :], z3_s[pl.ds(r0 + S3, 32), :])
        m2 = jnp.dot(eo3, vt, preferred_element_type=f32)
        m = jnp.maximum(m2[0:16], m2[16:32])
        feat_ref[0, pl.ds(py * HP, HP), :] = (
            jnp.maximum(m + b3, 0.0).astype(feat_ref.dtype))


def _conv_tower(p1, w1, b1, w2, b2, w3, b3, eo1, eo2, eo3):
    B = p1.shape[0]
    return pl.pallas_call(
        _tower_kernel,
        out_shape=jax.ShapeDtypeStruct((B, HP * HP, C3), jnp.bfloat16),
        grid=(2, B // 2),
        in_specs=[
            pl.BlockSpec((1, M1, 27), lambda i, j: (i * (B // 2) + j, 0, 0)),
            pl.BlockSpec((27, C1), lambda i, j: (0, 0)),
            pl.BlockSpec((1, C1), lambda i, j: (0, 0)),
            pl.BlockSpec((9 * C1, C2), lambda i, j: (0, 0)),
            pl.BlockSpec((1, C2), lambda i, j: (0, 0)),
            pl.BlockSpec((9 * C2, C3), lambda i, j: (0, 0)),
            pl.BlockSpec((1, C3), lambda i, j: (0, 0)),
            pl.BlockSpec((128, 128), lambda i, j: (0, 0)),
            pl.BlockSpec((64, 64), lambda i, j: (0, 0)),
            pl.BlockSpec((32, 32), lambda i, j: (0, 0)),
        ],
        out_specs=pl.BlockSpec((1, HP * HP, C3),
                               lambda i, j: (i * (B // 2) + j, 0, 0)),
        scratch_shapes=[
            pltpu.VMEM((M1, C1), jnp.float32),
            pltpu.VMEM((A2R, C1), jnp.bfloat16),
            pltpu.VMEM((M2, 9 * C1), jnp.bfloat16),
            pltpu.VMEM((M2, C2), jnp.float32),
            pltpu.VMEM((A3R, C2), jnp.bfloat16),
            pltpu.VMEM((M3, 9 * C2), jnp.bfloat16),
            pltpu.VMEM((M3, C3), jnp.float32),
        ],
        compiler_params=pltpu.CompilerParams(
            dimension_semantics=("parallel", "arbitrary")),
    )(p1, w1, b1, w2, b2, w3, b3, eo1, eo2, eo3)


def _mlp_kernel(x_ref, w1_ref, b1_ref, w2_ref, b2_ref, o_ref):
    h = jnp.dot(x_ref[...], w1_ref[...], preferred_element_type=jnp.float32)
    h = jnp.maximum(h + b1_ref[...], 0.0)
    o_ref[...] = (jnp.dot(h, w2_ref[...], preferred_element_type=jnp.float32)
                  + b2_ref[...])


def _classifier(flat, w1, b1, w2, b2):
    B, D = flat.shape
    Hd, C = w1.shape[1], w2.shape[1]
    mb = B // 2
    return pl.pallas_call(
        _mlp_kernel,
        out_shape=jax.ShapeDtypeStruct((B, C), jnp.float32),
        grid=(2,),
        in_specs=[pl.BlockSpec((mb, D), lambda i: (i, 0)),
                  pl.BlockSpec((D, Hd), lambda i: (0, 0)),
                  pl.BlockSpec((1, Hd), lambda i: (0, 0)),
                  pl.BlockSpec((Hd, C), lambda i: (0, 0)),
                  pl.BlockSpec((1, C), lambda i: (0, 0))],
        out_specs=pl.BlockSpec((mb, C), lambda i: (i, 0)),
        compiler_params=pltpu.CompilerParams(dimension_semantics=("parallel",)),
    )(flat, w1, b1, w2, b2)


def kernel(x_nchw, w1t, b1r, w2t, b2r, w3t, b3r,
           fc1_wt, fc1_br, fc2_wt, fc2_br, sel1, sel2, sel3):
    B = x_nchw.shape[0]
    bf16 = jnp.bfloat16

    x = jnp.transpose(x_nchw, (0, 2, 3, 1)).astype(bf16)
    xp = jnp.pad(x, ((0, 0), (1, 1), (1, 1), (0, 0)))
    cols = [xp[:, ky:ky + H, kx:kx + W, :] for ky in range(3) for kx in range(3)]
    p1 = jnp.concatenate(cols, axis=-1).reshape(B, M1, 27)

    w1 = w1t.astype(bf16)
    w2 = w2t.reshape(9 * C1, C2).astype(bf16)
    w3 = w3t.reshape(9 * C2, C3).astype(bf16)
    eo1 = sel1.reshape(128, 128)
    eo2 = sel2.reshape(64, 64)
    eo3 = sel3.reshape(32, 32)

    feats = _conv_tower(p1, w1, b1r, w2, b2r, w3, b3r, eo1, eo2, eo3)
    flat = feats.reshape(B, HP * HP * C3)
    return _classifier(flat, fc1_wt.astype(bf16), fc1_br, fc2_wt, fc2_br)
